# baseline (speedup 1.0000x reference)
import jax
import jax.numpy as jnp
from jax import lax
from jax.experimental import pallas as pl
from jax.experimental.pallas import tpu as pltpu

_CHUNK = 2048
_NBUF = 8


def _mlp_body(x_hbm, w1, b1, w2, b2, o_hbm, xb, ob, isem, osem):
    n = x_hbm.shape[0]
    nchunk = n // _CHUNK

    def in_copy(c, slot):
        return pltpu.make_async_copy(
            x_hbm.at[pl.ds(c * _CHUNK, _CHUNK), :], xb.at[slot], isem.at[slot])

    def out_copy(c, slot):
        return pltpu.make_async_copy(
            ob.at[slot], o_hbm.at[pl.ds(c * _CHUNK, _CHUNK), :], osem.at[slot])

    for s in range(_NBUF):
        in_copy(s, s).start()

    w1v = w1[...]
    b1v = b1[...]
    w2v = w2[...]
    b2v = b2[...]

    def outer(g, carry):
        for s in range(_NBUF):
            c = g * _NBUF + s
            in_copy(c, s).wait()
            x = xb.at[s][...]
            h = jnp.dot(x, w1v, preferred_element_type=jnp.float32) + b1v
            h = jnp.where(h >= 0, h, 0.01 * h)
            o = jnp.dot(h, w2v, preferred_element_type=jnp.float32) + b2v

            @pl.when(c >= _NBUF)
            def _():
                out_copy(c - _NBUF, s).wait()

            ob.at[s][...] = o
            out_copy(c, s).start()

            @pl.when(c + _NBUF < nchunk)
            def _():
                in_copy(c + _NBUF, s).start()
        return carry

    lax.fori_loop(0, nchunk // _NBUF, outer, 0)

    for s in range(_NBUF):
        out_copy(nchunk - _NBUF + s, s).wait()


def kernel(batch_x, W1, b1, W2, b2):
    n, d_in = batch_x.shape
    d_h = W1.shape[0]
    n_bins = W2.shape[0]

    return pl.pallas_call(
        _mlp_body,
        in_specs=[
            pl.BlockSpec(memory_space=pl.ANY),
            pl.BlockSpec(memory_space=pltpu.VMEM),
            pl.BlockSpec(memory_space=pltpu.VMEM),
            pl.BlockSpec(memory_space=pltpu.VMEM),
            pl.BlockSpec(memory_space=pltpu.VMEM),
        ],
        out_specs=pl.BlockSpec(memory_space=pl.ANY),
        out_shape=jax.ShapeDtypeStruct((n, n_bins), jnp.float32),
        scratch_shapes=[
            pltpu.VMEM((_NBUF, _CHUNK, d_in), jnp.float32),
            pltpu.VMEM((_NBUF, _CHUNK, n_bins), jnp.float32),
            pltpu.SemaphoreType.DMA((_NBUF,)),
            pltpu.SemaphoreType.DMA((_NBUF,)),
        ],
    )(batch_x, W1.T, b1.reshape(1, d_h), W2.T, b2.reshape(1, n_bins))
